# baseline (speedup 1.0000x reference)
import functools

import jax
import jax.numpy as jnp
from jax import lax
from jax.experimental import pallas as pl
from jax.experimental.pallas import tpu as pltpu
from jax.experimental.pallas import tpu_sc as plsc

N, E, D, O = 10000, 320000, 128, 128
EPS = 1e-5
NEG_SENTINEL = -1e37
NEG_INIT = -3.0e38

_BLK = 1000

NW = 32
ROWS = 313
DUMP = ROWS
MROWS = ROWS + 1
WE = 3200
NWIN = E // WE
SEG = 8
B = 480
FLUSH_AT = B - 16 * SEG


def _iota16():
    return lax.iota(jnp.int32, 16)


def _splat(scalar):
    return jnp.broadcast_to(scalar, (16,))


_GDN = lax.GatherDimensionNumbers(offset_dims=(), collapsed_slice_dims=(0,),
                                  start_index_map=(0,))


def _bcast_lane(vec, i):
    idx = jnp.full((16, 1), i, jnp.int32)
    return lax.gather(vec, idx, _GDN, (1,),
                      mode=lax.GatherScatterMode.PROMISE_IN_BOUNDS)


def _sc_body(y_hbm, dst_hbm, src_hbm, w_hbm, coef_hbm,
             m_out, stats_out,
             dst_v0, src_v0, w_v0, dst_v1, src_v1, w_v1,
             ldst_b, srcidx_b, s_b, rows_v, m_v, sq_v,
             coef_v, sem, sem0, sem1):
    wid = lax.axis_index("s") * 2 + lax.axis_index("c")
    lo = wid * ROWS
    it = _iota16()

    def _init_m(j, _):
        plsc.store_scatter(m_v, [j * 16 + it], jnp.full((16,), NEG_INIT,
                                                        jnp.float32))
        return 0
    lax.fori_loop(0, MROWS * D // 16, _init_m, 0)
    for g in range(16):
        sq_v[pl.ds(g * 16, 16)] = jnp.zeros((16,), jnp.float32)

    def _reset_bufs():
        for g in range(B // 16):
            ldst_b[pl.ds(g * 16, 16)] = jnp.full((16,), DUMP, jnp.int32)
            s_b[pl.ds(g * 16, 16)] = jnp.zeros((16,), jnp.float32)
            srcidx_b[pl.ds(g * 16, 16)] = jnp.full((16,), 8 * g, jnp.int32)
    _reset_bufs()

    pltpu.sync_copy(coef_hbm, coef_v)
    coef = coef_v[...]

    def _flush(_c):
        pltpu.async_copy(y_hbm.at[srcidx_b], rows_v, sem).wait()

        def _grp(g, carry):
            accs = carry
            lds = plsc.load_gather(ldst_b, [g * 16 + it])
            ss = plsc.load_gather(s_b, [g * 16 + it])
            for i in range(16):
                ld_i = _bcast_lane(lds, i)
                s_i = _bcast_lane(ss, i)
                e_i = _splat(g * 16 + i)
                base = ld_i * D
                new_accs = []
                for cg in range(8):
                    col = cg * 16 + it
                    row = plsc.load_gather(rows_v, [e_i, col])
                    q = s_i * row
                    addr = base + col
                    cur = plsc.load_gather(m_v, [addr])
                    plsc.store_scatter(m_v, [addr], jnp.maximum(cur, q))
                    new_accs.append(accs[cg] + q)
                    new_accs.append(accs[8 + cg] + q * q)
                accs = tuple(new_accs[0::2]) + tuple(new_accs[1::2])
            return accs

        zero = jnp.zeros((16,), jnp.float32)
        accs = lax.fori_loop(0, B // 16, _grp, (zero,) * 16)
        for cg in range(8):
            sq_v[pl.ds(cg * 16, 16)] += accs[cg]
            sq_v[pl.ds(D + cg * 16, 16)] += accs[8 + cg]
        _reset_bufs()
        return jnp.zeros((16,), jnp.int32)

    def _process(dbuf, sbuf, wbuf, cnt):
        def _seg(sg, c):
            for k in range(SEG):
                idx = (sg * SEG + k) * 16 + it
                d = plsc.load_gather(dbuf, [idx])
                sv = plsc.load_gather(sbuf, [idx])
                wv = plsc.load_gather(wbuf, [idx])
                ld = d - lo
                mask = (ld >= 0) & (ld < ROWS)
                s = 1.0 + coef * wv
                mi = mask.astype(jnp.int32)
                pos = plsc.cumsum(mi) + (c - 1)
                plsc.store_scatter(ldst_b, [pos], ld, mask=mask)
                plsc.store_scatter(srcidx_b, [pos], sv, mask=mask)
                plsc.store_scatter(s_b, [pos], s, mask=mask)
                c = c + plsc.all_reduce_population_count(mask)
            return lax.cond(c[0] >= FLUSH_AT, _flush, lambda cc: cc, c)
        return lax.fori_loop(0, WE // 16 // SEG, _seg, cnt)

    def _issue(w, dbuf, sbuf, wbuf, sm):
        base = jnp.minimum(w, NWIN - 1) * WE
        pltpu.async_copy(dst_hbm.at[pl.ds(base, WE)], dbuf, sm)
        pltpu.async_copy(src_hbm.at[pl.ds(base, WE)], sbuf, sm)
        pltpu.async_copy(w_hbm.at[pl.ds(base, WE)], wbuf, sm)

    def _wait(dbuf, sbuf, wbuf, sm):
        pltpu.make_async_copy(dst_hbm.at[pl.ds(0, WE)], dbuf, sm).wait()
        pltpu.make_async_copy(src_hbm.at[pl.ds(0, WE)], sbuf, sm).wait()
        pltpu.make_async_copy(w_hbm.at[pl.ds(0, WE)], wbuf, sm).wait()

    _issue(jnp.int32(0), dst_v0, src_v0, w_v0, sem0)

    def _wpair(p, cnt):
        w = p * 2
        _wait(dst_v0, src_v0, w_v0, sem0)
        _issue(w + 1, dst_v1, src_v1, w_v1, sem1)
        cnt = _process(dst_v0, src_v0, w_v0, cnt)
        _wait(dst_v1, src_v1, w_v1, sem1)
        _issue(w + 2, dst_v0, src_v0, w_v0, sem0)
        return _process(dst_v1, src_v1, w_v1, cnt)

    cnt = lax.fori_loop(0, NWIN // 2, _wpair, jnp.zeros((16,), jnp.int32))
    _wait(dst_v0, src_v0, w_v0, sem0)
    _flush(cnt)

    pltpu.sync_copy(m_v, m_out.at[wid])
    pltpu.sync_copy(sq_v, stats_out.at[wid])


def _sparse_stage(y, dst, src, w, coef_arr):
    mesh = plsc.VectorSubcoreMesh(core_axis_name="c", subcore_axis_name="s")
    fn = functools.partial(
        pl.kernel, mesh=mesh,
        compiler_params=pltpu.CompilerParams(needs_layout_passes=False),
        out_type=[
            jax.ShapeDtypeStruct((NW, MROWS * D), jnp.float32),
            jax.ShapeDtypeStruct((NW, 2 * D), jnp.float32),
        ],
        scratch_types=[
            pltpu.VMEM((WE,), jnp.int32),
            pltpu.VMEM((WE,), jnp.int32),
            pltpu.VMEM((WE,), jnp.float32),
            pltpu.VMEM((WE,), jnp.int32),
            pltpu.VMEM((WE,), jnp.int32),
            pltpu.VMEM((WE,), jnp.float32),
            pltpu.VMEM((B,), jnp.int32),
            pltpu.VMEM((B,), jnp.int32),
            pltpu.VMEM((B,), jnp.float32),
            pltpu.VMEM((B, D), jnp.float32),
            pltpu.VMEM((MROWS * D,), jnp.float32),
            pltpu.VMEM((2 * D,), jnp.float32),
            pltpu.VMEM((16,), jnp.float32),
            pltpu.SemaphoreType.DMA,
            pltpu.SemaphoreType.DMA,
            pltpu.SemaphoreType.DMA,
        ],
    )(_sc_body)
    return fn(y, dst, src, w, coef_arr)


def _mm_body(x_ref, w_ref, o_ref):
    o_ref[...] = jnp.dot(x_ref[...], w_ref[...],
                         preferred_element_type=jnp.float32)


def _matmul_pool(x, w):
    return pl.pallas_call(
        _mm_body,
        grid=(N // _BLK,),
        in_specs=[
            pl.BlockSpec((_BLK, D), lambda i: (i, 0)),
            pl.BlockSpec((D, D), lambda i: (0, 0)),
        ],
        out_specs=pl.BlockSpec((_BLK, D), lambda i: (i, 0)),
        out_shape=jax.ShapeDtypeStruct((N, D), jnp.float32),
    )(x, w)


def _mid_body(m_ref, stats_ref, x_ref, w1_ref, w2_ref, bpool_ref, gp_ref,
              bp_ref, bf_ref, h_ref, hstats_ref):
    i = pl.program_id(0)
    stats = stats_ref[...]
    s_sum = jnp.sum(stats[:, :D], axis=0, keepdims=True)
    q_sum = jnp.sum(stats[:, D:], axis=0, keepdims=True)
    b = bpool_ref[...]
    mu = s_sum / E + b
    mean_sq = q_sum / E + 2.0 * b * s_sum / E + b * b
    var = mean_sq - mu * mu
    inv = gp_ref[...] * jax.lax.rsqrt(var + EPS)
    m = m_ref[...]
    agg = jnp.maximum((m + b - mu) * inv + bp_ref[...], 0.0)
    agg = jnp.where(m > NEG_SENTINEL, agg, 0.0)
    h = (jnp.dot(x_ref[...], w1_ref[...], preferred_element_type=jnp.float32)
         + jnp.dot(agg, w2_ref[...], preferred_element_type=jnp.float32)
         + bf_ref[...])
    h_ref[...] = h
    part = jnp.concatenate([jnp.sum(h, axis=0, keepdims=True),
                            jnp.sum(h * h, axis=0, keepdims=True)], axis=0)

    @pl.when(i == 0)
    def _():
        hstats_ref[...] = jnp.zeros_like(hstats_ref)

    hstats_ref[...] += part


def _mid_stage(m, stats, x, w1, w2, b_pool, g_pool, beta_pool, b_final):
    k = stats.shape[0]
    return pl.pallas_call(
        _mid_body,
        grid=(N // _BLK,),
        in_specs=[
            pl.BlockSpec((_BLK, D), lambda i: (i, 0)),
            pl.BlockSpec((k, 2 * D), lambda i: (0, 0)),
            pl.BlockSpec((_BLK, D), lambda i: (i, 0)),
            pl.BlockSpec((D, O), lambda i: (0, 0)),
            pl.BlockSpec((D, O), lambda i: (0, 0)),
            pl.BlockSpec((1, D), lambda i: (0, 0)),
            pl.BlockSpec((1, D), lambda i: (0, 0)),
            pl.BlockSpec((1, D), lambda i: (0, 0)),
            pl.BlockSpec((1, O), lambda i: (0, 0)),
        ],
        out_specs=[
            pl.BlockSpec((_BLK, O), lambda i: (i, 0)),
            pl.BlockSpec((2, O), lambda i: (0, 0)),
        ],
        out_shape=[
            jax.ShapeDtypeStruct((N, O), jnp.float32),
            jax.ShapeDtypeStruct((2, O), jnp.float32),
        ],
    )(m, stats, x, w1, w2, b_pool, g_pool, beta_pool, b_final)


def _final_body(h_ref, hstats_ref, gf_ref, betaf_ref, o_ref):
    hs = hstats_ref[...]
    mu = hs[0:1, :] / N
    var = hs[1:2, :] / N - mu * mu
    inv = gf_ref[...] * jax.lax.rsqrt(var + EPS)
    o_ref[...] = jnp.maximum((h_ref[...] - mu) * inv + betaf_ref[...], 0.0)


def _final_stage(h, hstats, g_final, beta_final):
    return pl.pallas_call(
        _final_body,
        grid=(N // _BLK,),
        in_specs=[
            pl.BlockSpec((_BLK, O), lambda i: (i, 0)),
            pl.BlockSpec((2, O), lambda i: (0, 0)),
            pl.BlockSpec((1, O), lambda i: (0, 0)),
            pl.BlockSpec((1, O), lambda i: (0, 0)),
        ],
        out_specs=pl.BlockSpec((_BLK, O), lambda i: (i, 0)),
        out_shape=jax.ShapeDtypeStruct((N, O), jnp.float32),
    )(h, hstats, g_final, beta_final)


def kernel(x, edge_index, edge_weight, W_pool, b_pool, bn_pool_gamma,
           bn_pool_beta, W_final, b_final, bn_final_gamma, bn_final_beta,
           edge_coef):
    src = edge_index[0]
    dst = edge_index[1]
    y = _matmul_pool(x, W_pool)

    coef_arr = jnp.full((16,), edge_coef, jnp.float32)
    m_parts, stats = _sparse_stage(y, dst, src, edge_weight, coef_arr)
    m = m_parts.reshape(NW, MROWS, D)[:, :ROWS, :].reshape(NW * ROWS, D)[:N]

    w1 = W_final[:D, :]
    w2 = W_final[D:, :]
    h, hstats = _mid_stage(
        m, stats, x, w1, w2,
        b_pool.reshape(1, D), bn_pool_gamma.reshape(1, D),
        bn_pool_beta.reshape(1, D), b_final.reshape(1, O))
    return _final_stage(h, hstats, bn_final_gamma.reshape(1, O),
                        bn_final_beta.reshape(1, O))


# speedup vs baseline: 1.0152x; 1.0152x over previous
import functools

import jax
import jax.numpy as jnp
from jax import lax
from jax.experimental import pallas as pl
from jax.experimental.pallas import tpu as pltpu
from jax.experimental.pallas import tpu_sc as plsc

N, E, D, O = 10000, 320000, 128, 128
EPS = 1e-5
NEG_SENTINEL = -1e37
NEG_INIT = -3.0e38

_BLK = 1000

NW = 32
ROWS = 313
DUMP = ROWS
MROWS = ROWS + 1
WE = 6400
NWIN = E // WE
SEG = 8
B = 480
FLUSH_AT = B - 16 * SEG


def _iota16():
    return lax.iota(jnp.int32, 16)


def _splat(scalar):
    return jnp.broadcast_to(scalar, (16,))


_GDN = lax.GatherDimensionNumbers(offset_dims=(), collapsed_slice_dims=(0,),
                                  start_index_map=(0,))


def _bcast_lane(vec, i):
    idx = jnp.full((16, 1), i, jnp.int32)
    return lax.gather(vec, idx, _GDN, (1,),
                      mode=lax.GatherScatterMode.PROMISE_IN_BOUNDS)


def _sc_body(y_hbm, dst_hbm, src_hbm, s_hbm,
             m_out, stats_out,
             dst_v0, dst_v1,
             eid_b, ldst_b, srcv_b, sg_b, rows_v, m_v, sq_v,
             sem, sem0, sem1):
    wid = lax.axis_index("s") * 2 + lax.axis_index("c")
    lo = wid * ROWS
    it = _iota16()

    def _init_m(j, _):
        plsc.store_scatter(m_v, [j * 16 + it], jnp.full((16,), NEG_INIT,
                                                        jnp.float32))
        return 0
    lax.fori_loop(0, MROWS * D // 16, _init_m, 0)
    for g in range(16):
        sq_v[pl.ds(g * 16, 16)] = jnp.zeros((16,), jnp.float32)

    def _reset_bufs():
        for g in range(B // 16):
            ldst_b[pl.ds(g * 16, 16)] = jnp.full((16,), DUMP, jnp.int32)
            eid_b[pl.ds(g * 16, 16)] = wid * B + g * 16 + it
    _reset_bufs()

    def _flush(c):
        d1 = pltpu.async_copy(src_hbm.at[eid_b], srcv_b, sem)
        d2 = pltpu.async_copy(s_hbm.at[eid_b], sg_b, sem)
        d1.wait()
        d2.wait()
        pltpu.async_copy(y_hbm.at[srcv_b], rows_v, sem).wait()

        def _grp(g, carry):
            accs = carry
            lds = plsc.load_gather(ldst_b, [g * 16 + it])
            sraw = plsc.load_gather(sg_b, [g * 16 + it])
            ss = jnp.where(g * 16 + it < c, sraw, 0.0)
            for i in range(16):
                ld_i = _bcast_lane(lds, i)
                s_i = _bcast_lane(ss, i)
                e_i = _splat(g * 16 + i)
                base = ld_i * D
                new_accs = []
                for cg in range(8):
                    col = cg * 16 + it
                    row = plsc.load_gather(rows_v, [e_i, col])
                    q = s_i * row
                    addr = base + col
                    cur = plsc.load_gather(m_v, [addr])
                    plsc.store_scatter(m_v, [addr], jnp.maximum(cur, q))
                    new_accs.append(accs[cg] + q)
                    new_accs.append(accs[8 + cg] + q * q)
                accs = tuple(new_accs[0::2]) + tuple(new_accs[1::2])
            return accs

        zero = jnp.zeros((16,), jnp.float32)
        accs = lax.fori_loop(0, B // 16, _grp, (zero,) * 16)
        for cg in range(8):
            sq_v[pl.ds(cg * 16, 16)] += accs[cg]
            sq_v[pl.ds(D + cg * 16, 16)] += accs[8 + cg]
        _reset_bufs()
        return jnp.zeros((16,), jnp.int32)

    def _process(dbuf, ebase, cnt):
        def _seg(sg, c):
            for k in range(SEG):
                idx = (sg * SEG + k) * 16 + it
                d = plsc.load_gather(dbuf, [idx])
                ld = d - lo
                mask = (ld >= 0) & (ld < ROWS)
                mi = mask.astype(jnp.int32)
                pos = plsc.cumsum(mi) + (c - 1)
                plsc.store_scatter(ldst_b, [pos], ld, mask=mask)
                plsc.store_scatter(eid_b, [pos], ebase + idx, mask=mask)
                c = c + plsc.all_reduce_population_count(mask)
            return lax.cond(c[0] >= FLUSH_AT, _flush, lambda cc: cc, c)
        return lax.fori_loop(0, WE // 16 // SEG, _seg, cnt)

    def _issue(w, dbuf, sm):
        base = jnp.minimum(w, NWIN - 1) * WE
        pltpu.async_copy(dst_hbm.at[pl.ds(base, WE)], dbuf, sm)

    def _wait(dbuf, sm):
        pltpu.make_async_copy(dst_hbm.at[pl.ds(0, WE)], dbuf, sm).wait()

    _issue(jnp.int32(0), dst_v0, sem0)

    def _wpair(p, cnt):
        w = p * 2
        _wait(dst_v0, sem0)
        _issue(w + 1, dst_v1, sem1)
        cnt = _process(dst_v0, w * WE, cnt)
        _wait(dst_v1, sem1)
        _issue(w + 2, dst_v0, sem0)
        return _process(dst_v1, (w + 1) * WE, cnt)

    cnt = lax.fori_loop(0, NWIN // 2, _wpair, jnp.zeros((16,), jnp.int32))
    _wait(dst_v0, sem0)
    _flush(cnt)

    pltpu.sync_copy(m_v, m_out.at[wid])
    pltpu.sync_copy(sq_v, stats_out.at[wid])


def _sparse_stage(y, dst, src, s_all):
    mesh = plsc.VectorSubcoreMesh(core_axis_name="c", subcore_axis_name="s")
    fn = functools.partial(
        pl.kernel, mesh=mesh,
        compiler_params=pltpu.CompilerParams(needs_layout_passes=False),
        out_type=[
            jax.ShapeDtypeStruct((NW, MROWS * D), jnp.float32),
            jax.ShapeDtypeStruct((NW, 2 * D), jnp.float32),
        ],
        scratch_types=[
            pltpu.VMEM((WE,), jnp.int32),
            pltpu.VMEM((WE,), jnp.int32),
            pltpu.VMEM((B,), jnp.int32),
            pltpu.VMEM((B,), jnp.int32),
            pltpu.VMEM((B,), jnp.int32),
            pltpu.VMEM((B,), jnp.float32),
            pltpu.VMEM((B, D), jnp.float32),
            pltpu.VMEM((MROWS * D,), jnp.float32),
            pltpu.VMEM((2 * D,), jnp.float32),
            pltpu.SemaphoreType.DMA,
            pltpu.SemaphoreType.DMA,
            pltpu.SemaphoreType.DMA,
        ],
    )(_sc_body)
    return fn(y, dst, src, s_all)


_EROWS = E // D
_EBLK = _EROWS // (N // _BLK)


def _mm_body(x_ref, w_ref, ew_ref, coef_ref, o_ref, s_ref):
    o_ref[...] = jnp.dot(x_ref[...], w_ref[...],
                         preferred_element_type=jnp.float32)

    @pl.when(pl.program_id(0) == 0)
    def _():
        s_ref[...] = 1.0 + coef_ref[...] * ew_ref[...]


def _matmul_pool(x, w, ew2d, coef2d):
    return pl.pallas_call(
        _mm_body,
        grid=(N // _BLK,),
        in_specs=[
            pl.BlockSpec((_BLK, D), lambda i: (i, 0)),
            pl.BlockSpec((D, D), lambda i: (0, 0)),
            pl.BlockSpec((_EROWS, D), lambda i: (0, 0)),
            pl.BlockSpec((1, D), lambda i: (0, 0)),
        ],
        out_specs=[
            pl.BlockSpec((_BLK, D), lambda i: (i, 0)),
            pl.BlockSpec((_EROWS, D), lambda i: (0, 0)),
        ],
        out_shape=[
            jax.ShapeDtypeStruct((N, D), jnp.float32),
            jax.ShapeDtypeStruct((_EROWS, D), jnp.float32),
        ],
    )(x, w, ew2d, coef2d)


def _mid_body(m_ref, stats_ref, x_ref, w1_ref, w2_ref, bpool_ref, gp_ref,
              bp_ref, bf_ref, h_ref, hstats_ref):
    i = pl.program_id(0)
    stats = stats_ref[...]
    s_sum = jnp.sum(stats[:, :D], axis=0, keepdims=True)
    q_sum = jnp.sum(stats[:, D:], axis=0, keepdims=True)
    b = bpool_ref[...]
    mu = s_sum / E + b
    mean_sq = q_sum / E + 2.0 * b * s_sum / E + b * b
    var = mean_sq - mu * mu
    inv = gp_ref[...] * jax.lax.rsqrt(var + EPS)
    m = m_ref[...]
    agg = jnp.maximum((m + b - mu) * inv + bp_ref[...], 0.0)
    agg = jnp.where(m > NEG_SENTINEL, agg, 0.0)
    h = (jnp.dot(x_ref[...], w1_ref[...], preferred_element_type=jnp.float32)
         + jnp.dot(agg, w2_ref[...], preferred_element_type=jnp.float32)
         + bf_ref[...])
    h_ref[...] = h
    part = jnp.concatenate([jnp.sum(h, axis=0, keepdims=True),
                            jnp.sum(h * h, axis=0, keepdims=True)], axis=0)

    @pl.when(i == 0)
    def _():
        hstats_ref[...] = jnp.zeros_like(hstats_ref)

    hstats_ref[...] += part


def _mid_stage(m, stats, x, w1, w2, b_pool, g_pool, beta_pool, b_final):
    k = stats.shape[0]
    return pl.pallas_call(
        _mid_body,
        grid=(N // _BLK,),
        in_specs=[
            pl.BlockSpec((_BLK, D), lambda i: (i, 0)),
            pl.BlockSpec((k, 2 * D), lambda i: (0, 0)),
            pl.BlockSpec((_BLK, D), lambda i: (i, 0)),
            pl.BlockSpec((D, O), lambda i: (0, 0)),
            pl.BlockSpec((D, O), lambda i: (0, 0)),
            pl.BlockSpec((1, D), lambda i: (0, 0)),
            pl.BlockSpec((1, D), lambda i: (0, 0)),
            pl.BlockSpec((1, D), lambda i: (0, 0)),
            pl.BlockSpec((1, O), lambda i: (0, 0)),
        ],
        out_specs=[
            pl.BlockSpec((_BLK, O), lambda i: (i, 0)),
            pl.BlockSpec((2, O), lambda i: (0, 0)),
        ],
        out_shape=[
            jax.ShapeDtypeStruct((N, O), jnp.float32),
            jax.ShapeDtypeStruct((2, O), jnp.float32),
        ],
    )(m, stats, x, w1, w2, b_pool, g_pool, beta_pool, b_final)


def _final_body(h_ref, hstats_ref, gf_ref, betaf_ref, o_ref):
    hs = hstats_ref[...]
    mu = hs[0:1, :] / N
    var = hs[1:2, :] / N - mu * mu
    inv = gf_ref[...] * jax.lax.rsqrt(var + EPS)
    o_ref[...] = jnp.maximum((h_ref[...] - mu) * inv + betaf_ref[...], 0.0)


def _final_stage(h, hstats, g_final, beta_final):
    return pl.pallas_call(
        _final_body,
        grid=(N // _BLK,),
        in_specs=[
            pl.BlockSpec((_BLK, O), lambda i: (i, 0)),
            pl.BlockSpec((2, O), lambda i: (0, 0)),
            pl.BlockSpec((1, O), lambda i: (0, 0)),
            pl.BlockSpec((1, O), lambda i: (0, 0)),
        ],
        out_specs=pl.BlockSpec((_BLK, O), lambda i: (i, 0)),
        out_shape=jax.ShapeDtypeStruct((N, O), jnp.float32),
    )(h, hstats, g_final, beta_final)


def kernel(x, edge_index, edge_weight, W_pool, b_pool, bn_pool_gamma,
           bn_pool_beta, W_final, b_final, bn_final_gamma, bn_final_beta,
           edge_coef):
    src = edge_index[0]
    dst = edge_index[1]
    coef2d = jnp.full((1, D), edge_coef, jnp.float32)
    y, s2d = _matmul_pool(x, W_pool, edge_weight.reshape(_EROWS, D), coef2d)
    m_parts, stats = _sparse_stage(y, dst, src, s2d.reshape(E))
    m = m_parts.reshape(NW, MROWS, D)[:, :ROWS, :].reshape(NW * ROWS, D)[:N]

    w1 = W_final[:D, :]
    w2 = W_final[D:, :]
    h, hstats = _mid_stage(
        m, stats, x, w1, w2,
        b_pool.reshape(1, D), bn_pool_gamma.reshape(1, D),
        bn_pool_beta.reshape(1, D), b_final.reshape(1, O))
    return _final_stage(h, hstats, bn_final_gamma.reshape(1, O),
                        bn_final_beta.reshape(1, O))
